# Initial kernel scaffold; baseline (speedup 1.0000x reference)
import jax
import jax.numpy as jnp
from jax.experimental import pallas as pl


def kernel(queries, labels, heads, proposals, table):
    raise NotImplementedError("write your pallas kernel here")


# speedup vs baseline: 17.6591x; 17.6591x over previous
import functools

import jax
import jax.numpy as jnp
from jax import lax
from jax.experimental import pallas as pl
from jax.experimental.pallas import tpu as pltpu
from jax.experimental.pallas import tpu_sc as plsc

B, Q, W, P, D = 8, 32, 16, 64, 64
L = 16
PC = P // 4
ROWS = PC * Q
GCH = 128


def _splat(x):
    return x if getattr(x, "shape", ()) == (L,) else jnp.full((L,), x, jnp.int32)


@functools.cache
def _build_sc_encode():
    @functools.partial(
        pl.kernel,
        mesh=plsc.VectorSubcoreMesh(core_axis_name="c", subcore_axis_name="s"),
        compiler_params=pltpu.CompilerParams(
            needs_layout_passes=False, use_tc_tiling_on_sc=False),
        out_type=jax.ShapeDtypeStruct((B * P * Q, D), jnp.float32),
        scratch_types=[
            pltpu.VMEM((Q * W,), jnp.int32),
            pltpu.VMEM((PC,), jnp.int32),
            pltpu.VMEM((Q * 4,), jnp.int32),
            pltpu.VMEM((PC * 4,), jnp.float32),
            pltpu.VMEM((ROWS // GCH, GCH), jnp.int32),
            pltpu.VMEM((ROWS,), jnp.float32),
            pltpu.VMEM((ROWS, D), jnp.float32),
            pltpu.SemaphoreType.DMA,
        ],
    )
    def _sc_encode(q_hbm, l_hbm, h_hbm, pr_hbm, table_hbm, out_hbm,
                   q_v, l_v, h_v, pr_v, idx_v, keep_v, rows_v, sem):
        wid = lax.axis_index("s") * 2 + lax.axis_index("c")
        b = wid // 4
        p0 = (wid % 4) * PC

        pltpu.sync_copy(q_hbm.at[pl.ds(b * Q * W, Q * W)], q_v)
        pltpu.sync_copy(l_hbm.at[pl.ds(b * P + p0, PC)], l_v)
        pltpu.sync_copy(h_hbm.at[pl.ds(b * Q * 4, Q * 4)], h_v)
        pltpu.sync_copy(pr_hbm.at[pl.ds((b * P + p0) * 4, PC * 4)], pr_v)

        lanes = lax.iota(jnp.int32, L)

        prop = pr_v[pl.ds(0, L)] != 0.0
        for c4 in range(1, 4):
            prop = prop | (plsc.load_gather(pr_v, [lanes * 4 + c4]) != 0.0)

        lab = l_v[...]

        anybad = jnp.bool_(False)
        hch = None
        for q in range(Q):
            row = q_v[pl.ds(q * W, W)]
            if q % 4 == 0:
                hch = h_v[pl.ds((q // 4) * L, L)]
            head = hch[(q % 4) * 4]
            is_word = row != 0
            isq = plsc.all_reduce_population_count(is_word) > 0
            is_head = (row == head) & isq
            masked = jnp.where(is_head, row, 0)
            mx = jnp.max(masked)
            wi = _splat(plsc.all_reduce_ffs(masked == mx))
            wsel = plsc.load_gather(q_v, [wi + q * W])
            hsel = (wsel == head) & isq
            valid = (wsel != 0) & isq
            idx_p = jnp.where(hsel, lab, wsel)
            keep = valid & prop
            pos = lanes * Q + q
            plsc.store_scatter(idx_v, [pos >> 7, pos & (GCH - 1)], idx_p)
            plsc.store_scatter(keep_v, [pos], keep.astype(jnp.float32))
            anybad = anybad | jnp.any(~keep)

        copies = [
            pltpu.async_copy(table_hbm.at[idx_v.at[j]],
                             rows_v.at[pl.ds(j * GCH, GCH)], sem)
            for j in range(ROWS // GCH)
        ]
        for cp in copies:
            cp.wait()

        @pl.when(anybad)
        def _fix():
            def body(r, carry):
                sc = plsc.load_gather(keep_v, [_splat(r)])
                for c4 in range(D // L):
                    col = lanes + c4 * L
                    v = plsc.load_gather(rows_v, [_splat(r), col])
                    plsc.store_scatter(rows_v, [_splat(r), col], v * sc)
                return carry
            lax.fori_loop(0, ROWS, body, 0)

        pltpu.sync_copy(rows_v, out_hbm.at[pl.ds(wid * ROWS, ROWS)])

    return _sc_encode


def kernel(queries, labels, heads, proposals, table):
    qf = queries.astype(jnp.int32).reshape(-1)
    lf = labels.astype(jnp.int32).reshape(-1)
    hf = heads.astype(jnp.int32).reshape(-1)
    pf = proposals.astype(jnp.float32).reshape(-1)
    out = _build_sc_encode()(qf, lf, hf, pf, table.astype(jnp.float32))
    return out.reshape(B, P, Q, D)


# speedup vs baseline: 17.8681x; 1.0118x over previous
import functools

import jax
import jax.numpy as jnp
from jax import lax
from jax.experimental import pallas as pl
from jax.experimental.pallas import tpu as pltpu
from jax.experimental.pallas import tpu_sc as plsc

B, Q, W, P, D = 8, 32, 16, 64, 64
L = 16
PC = P // 4
ROWS = PC * Q
GCH = 128


def _splat(x):
    return x if getattr(x, "shape", ()) == (L,) else jnp.full((L,), x, jnp.int32)


@functools.cache
def _build_sc_encode():
    @functools.partial(
        pl.kernel,
        mesh=plsc.VectorSubcoreMesh(core_axis_name="c", subcore_axis_name="s"),
        compiler_params=pltpu.CompilerParams(
            needs_layout_passes=False, use_tc_tiling_on_sc=False),
        out_type=jax.ShapeDtypeStruct((B * P * Q, D), jnp.float32),
        scratch_types=[
            pltpu.VMEM((Q * W,), jnp.int32),
            pltpu.VMEM((PC,), jnp.int32),
            pltpu.VMEM((Q * 4,), jnp.int32),
            pltpu.VMEM((PC * 4,), jnp.float32),
            pltpu.VMEM((ROWS // GCH, GCH), jnp.int32),
            pltpu.VMEM((ROWS,), jnp.float32),
            pltpu.VMEM((ROWS, D), jnp.float32),
            pltpu.SemaphoreType.DMA,
            pltpu.SemaphoreType.DMA,
            pltpu.SemaphoreType.DMA,
            pltpu.SemaphoreType.DMA,
            pltpu.SemaphoreType.DMA,
            pltpu.SemaphoreType.DMA,
        ],
    )
    def _sc_encode(q_hbm, l_hbm, h_hbm, pr_hbm, table_hbm, out_hbm,
                   q_v, l_v, h_v, pr_v, idx_v, keep_v, rows_v,
                   sem_in, g0, g1, g2, g3, sem_out):
        wid = lax.axis_index("s") * 2 + lax.axis_index("c")
        b = wid // 4
        p0 = (wid % 4) * PC

        ins = [
            pltpu.async_copy(q_hbm.at[pl.ds(b * Q * W, Q * W)], q_v, sem_in),
            pltpu.async_copy(l_hbm.at[pl.ds(b * P + p0, PC)], l_v, sem_in),
            pltpu.async_copy(h_hbm.at[pl.ds(b * Q * 4, Q * 4)], h_v, sem_in),
            pltpu.async_copy(pr_hbm.at[pl.ds((b * P + p0) * 4, PC * 4)],
                             pr_v, sem_in),
        ]
        for cp in ins:
            cp.wait()

        lanes = lax.iota(jnp.int32, L)

        prop = pr_v[pl.ds(0, L)] != 0.0
        for c4 in range(1, 4):
            prop = prop | (plsc.load_gather(pr_v, [lanes * 4 + c4]) != 0.0)

        lab = l_v[...]

        anybad = jnp.bool_(False)
        hch = None
        for q in range(Q):
            row = q_v[pl.ds(q * W, W)]
            if q % 4 == 0:
                hch = h_v[pl.ds((q // 4) * L, L)]
            head = hch[(q % 4) * 4]
            is_word = row != 0
            isq = plsc.all_reduce_population_count(is_word) > 0
            is_head = (row == head) & isq
            masked = jnp.where(is_head, row, 0)
            mx = jnp.max(masked)
            wi = _splat(plsc.all_reduce_ffs(masked == mx))
            wsel = plsc.load_gather(q_v, [wi + q * W])
            hsel = (wsel == head) & isq
            valid = (wsel != 0) & isq
            idx_p = jnp.where(hsel, lab, wsel)
            keep = valid & prop
            pos = lanes * Q + q
            plsc.store_scatter(idx_v, [pos >> 7, pos & (GCH - 1)], idx_p)
            plsc.store_scatter(keep_v, [pos], keep.astype(jnp.float32))
            anybad = anybad | jnp.any(~keep)

        gsems = [g0, g1, g2, g3]
        gathers = [
            pltpu.async_copy(table_hbm.at[idx_v.at[j]],
                             rows_v.at[pl.ds(j * GCH, GCH)], gsems[j])
            for j in range(ROWS // GCH)
        ]
        outs = []
        for j in range(ROWS // GCH):
            gathers[j].wait()

            @pl.when(anybad)
            def _fix(j=j):
                def body(r, carry):
                    sc = plsc.load_gather(keep_v, [_splat(r)])
                    for c4 in range(D // L):
                        col = lanes + c4 * L
                        v = plsc.load_gather(rows_v, [_splat(r), col])
                        plsc.store_scatter(rows_v, [_splat(r), col], v * sc)
                    return carry
                lax.fori_loop(j * GCH, (j + 1) * GCH, body, 0)

            outs.append(pltpu.async_copy(
                rows_v.at[pl.ds(j * GCH, GCH)],
                out_hbm.at[pl.ds(wid * ROWS + j * GCH, GCH)], sem_out))
        for cp in outs:
            cp.wait()

    return _sc_encode


def kernel(queries, labels, heads, proposals, table):
    qf = queries.astype(jnp.int32).reshape(-1)
    lf = labels.astype(jnp.int32).reshape(-1)
    hf = heads.astype(jnp.int32).reshape(-1)
    pf = proposals.astype(jnp.float32).reshape(-1)
    out = _build_sc_encode()(qf, lf, hf, pf, table.astype(jnp.float32))
    return out.reshape(B, P, Q, D)
